# baseline (speedup 1.0000x reference)
import functools

import jax
import jax.numpy as jnp
from jax import lax
from jax.experimental import pallas as pl
from jax.experimental.pallas import tpu as pltpu
from jax.experimental.pallas import tpu_sc as plsc

_B, _P, _L, _F = 8, 24, 48, 768
_NCLS = 4
_NPAIR = _B * _P
_NW = 32
_PPW = _NPAIR // _NW
_WIN = 64
_3F = 3 * _F


def _sc_encode(tbl_flat, pairs_flat):
    mesh = plsc.VectorSubcoreMesh(core_axis_name="c", subcore_axis_name="s")

    @functools.partial(
        pl.kernel,
        out_type=jax.ShapeDtypeStruct((_NW, _PPW, _3F), jnp.float32),
        mesh=mesh,
        scratch_types=[
            pltpu.VMEM((_NPAIR * 4 + 16,), jnp.int32),
            pltpu.VMEM((_PPW, _WIN), jnp.int32),
            pltpu.VMEM((_WIN, _F), jnp.float32),
            pltpu.VMEM((_WIN, _F), jnp.float32),
            pltpu.VMEM((_PPW, _3F), jnp.float32),
            pltpu.SemaphoreType.DMA,
            pltpu.SemaphoreType.DMA,
        ],
    )
    def enc(tbl_hbm, pairs_hbm, out_hbm,
            pairs_v, idx_v, rows0, rows1, feats_v, sem0, sem1):
        wid = lax.axis_index("s") * 2 + lax.axis_index("c")
        base = wid * _PPW
        pltpu.sync_copy(pairs_hbm, pairs_v.at[pl.ds(0, _NPAIR * 4)])

        lanes = lax.iota(jnp.int32, 16)

        def pair_scalars(i):
            g = base + i
            pv = pairs_v[pl.ds(g * 4, 16)]
            s0, e0, s1, e1 = pv[0], pv[1], pv[2], pv[3]
            n = (e0 - s0) * (e1 - s1)
            return g, s0, e0, s1, e1, n

        for i in range(_PPW):
            g, s0, e0, s1, e1, n = pair_scalars(i)
            a = s0 + 1
            c = s1 + 1
            w = e1 - s1
            rowbase = (g // _P) * (_L * _L)
            eidx = rowbase + e0 * _L + e1
            for j in range(_WIN // 16):
                kv = lanes + (j * 16)
                kq = jnp.minimum(jnp.maximum(kv - 1, 0), n - 1)
                q = lax.div(kq, jnp.full((16,), 1, jnp.int32) * w)
                rm = kq - q * w
                idx = rowbase + (a + q) * _L + (c + rm)
                if j == 0:
                    idx = jnp.where(lanes == 0, eidx, idx)
                idx_v[i, pl.ds(j * 16, 16)] = idx

        bufs = (rows0, rows1)
        sems = (sem0, sem1)

        def issue(i, b):
            n = pair_scalars(i)[5]
            pltpu.async_copy(tbl_hbm.at[idx_v.at[i, pl.ds(0, 16)]],
                             bufs[b].at[pl.ds(0, 16)], sems[b])
            for j in range(1, _WIN // 16):
                @pl.when(j * 16 <= n)
                def _(j=j, b=b, i=i):
                    pltpu.async_copy(
                        tbl_hbm.at[idx_v.at[i, pl.ds(j * 16, 16)]],
                        bufs[b].at[pl.ds(j * 16, 16)], sems[b])

        def drain_chunk(i, b, j):
            pltpu.make_async_copy(
                tbl_hbm.at[idx_v.at[i, pl.ds(j * 16, 16)]],
                bufs[b].at[pl.ds(j * 16, 16)], sems[b]).wait()

        issue(0, 0)
        for i in range(_PPW):
            if i + 1 < _PPW:
                issue(i + 1, (i + 1) % 2)
            b = i % 2
            rows = bufs[b]
            n = pair_scalars(i)[5]

            def tree16(vs):
                m = list(vs)
                while len(m) > 1:
                    m = [jnp.maximum(m[2 * t], m[2 * t + 1])
                         for t in range(len(m) // 2)] + \
                        ([m[-1]] if len(m) % 2 else [])
                return m[0]

            drain_chunk(i, b, 0)

            def first_body(cc, carry, rows=rows, i=i):
                o = cc * 16
                vs = [rows[r, pl.ds(o, 16)] for r in range(16)]
                m0 = tree16(vs)
                feats_v[i, pl.ds(o, 16)] = vs[1]
                feats_v[i, pl.ds(_F + o, 16)] = vs[0]
                feats_v[i, pl.ds(2 * _F + o, 16)] = m0
                return carry

            lax.fori_loop(0, _F // 16, first_body, 0)

            for j in range(1, _WIN // 16):
                @pl.when(j * 16 <= n)
                def _(j=j, b=b, i=i, rows=rows):
                    drain_chunk(i, b, j)

                    def bb(cc, carry):
                        o = cc * 16
                        mj = tree16([rows[j * 16 + r, pl.ds(o, 16)]
                                     for r in range(16)])
                        feats_v[i, pl.ds(2 * _F + o, 16)] = jnp.maximum(
                            feats_v[i, pl.ds(2 * _F + o, 16)], mj)
                        return carry

                    lax.fori_loop(0, _F // 16, bb, 0)
        pltpu.sync_copy(feats_v, out_hbm.at[wid])

    return enc(tbl_flat, pairs_flat)


def _head(feats3, W, b2, labels3):
    def body(f_ref, w_ref, b_ref, l_ref, loss_ref, probs_ref, pred_ref):
        f = f_ref[...]
        w = w_ref[...]
        logits = lax.dot_general(f, w, (((2,), (1,)), ((), ())),
                                 preferred_element_type=jnp.float32)
        logits = logits + b_ref[...]
        m = jnp.max(logits, axis=-1, keepdims=True)
        ex = jnp.exp(logits - m)
        se = jnp.sum(ex, axis=-1, keepdims=True)
        probs_ref[...] = ex / se
        lbl = l_ref[...]
        cls = lax.broadcasted_iota(jnp.int32, (_NW, _PPW, _NCLS), 2)
        onehot = cls == lbl
        logp = (logits - m) - jnp.log(se)
        nll = -jnp.sum(jnp.where(onehot, logp, 0.0), axis=-1, keepdims=True)
        validf = (lbl >= 0).astype(jnp.float32)
        total = jnp.sum(nll * validf)
        cnt = jnp.maximum(jnp.sum(validf), 1.0)
        loss_ref[...] = (total / cnt) * jnp.ones((1, 1), jnp.float32)
        cand = jnp.where(logits == m, cls, _NCLS)
        pred_ref[...] = jnp.min(cand, axis=-1, keepdims=True)

    return pl.pallas_call(
        body,
        out_shape=(
            jax.ShapeDtypeStruct((1, 1), jnp.float32),
            jax.ShapeDtypeStruct((_NW, _PPW, _NCLS), jnp.float32),
            jax.ShapeDtypeStruct((_NW, _PPW, 1), jnp.int32),
        ),
    )(feats3, W, b2, labels3)


def kernel(Table, ia_seq, pairs, labels, W, b):
    del ia_seq
    tbl_flat = Table.reshape(_B * _L * _L, _F)
    pairs_flat = pairs.reshape(-1).astype(jnp.int32)
    feats3 = _sc_encode(tbl_flat, pairs_flat)
    loss11, probs, pred = _head(
        feats3,
        W.astype(jnp.float32),
        b.reshape(1, 1, _NCLS).astype(jnp.float32),
        labels.reshape(_NW, _PPW, 1).astype(jnp.int32),
    )
    return (loss11[0, 0], probs.reshape(_B, _P, _NCLS),
            pred.reshape(_B, _P).astype(jnp.int32))
